# baseline (speedup 1.0000x reference)
import functools

import jax
import jax.numpy as jnp
from jax.experimental import pallas as pl
from jax.experimental.pallas import tpu as pltpu


def _pool_body(S, B, mask_ref, enc_hbm, out_ref, sem):
    copies = []
    for b in range(B):
        total = jnp.sum(mask_ref[b, :].astype(jnp.int32))
        idx = jnp.clip(total - 1, 0, S - 1)
        copies.append(
            pltpu.make_async_copy(
                enc_hbm.at[b, pl.ds(idx, 1)],
                out_ref.at[pl.ds(b, 1)],
                sem.at[b],
            )
        )
        copies[-1].start()
    for c in copies:
        c.wait()


@functools.lru_cache(maxsize=None)
def _build_kernel(B: int, S: int, D: int):
    return pl.pallas_call(
        functools.partial(_pool_body, S, B),
        grid=(),
        in_specs=[
            pl.BlockSpec(memory_space=pltpu.VMEM),
            pl.BlockSpec(memory_space=pl.ANY),
        ],
        out_specs=pl.BlockSpec(memory_space=pl.ANY),
        out_shape=jax.ShapeDtypeStruct((B, D), jnp.float32),
        scratch_shapes=[pltpu.SemaphoreType.DMA((B,))],
    )


@jax.jit
def kernel(encoded_inputs, input_masks):
    B, S, D = encoded_inputs.shape
    return _build_kernel(B, S, D)(input_masks, encoded_inputs)


# speedup vs baseline: 1.2408x; 1.2408x over previous
import functools

import jax
import jax.numpy as jnp
from jax.experimental import pallas as pl
from jax.experimental.pallas import tpu as pltpu


def _pool_body(S, B, mask_ref, enc_hbm, out_ref, sem):
    copies = []
    for b in range(B):
        total = jnp.sum(mask_ref[b, :].astype(jnp.int32))
        idx = jnp.clip(total - 1, 0, S - 1)
        copies.append(
            pltpu.make_async_copy(
                enc_hbm.at[b, pl.ds(idx, 1)],
                out_ref.at[pl.ds(b, 1)],
                sem.at[b],
            )
        )
        copies[-1].start()
    for c in copies:
        c.wait()


@functools.lru_cache(maxsize=None)
def _build_kernel(B: int, S: int, D: int):
    return pl.pallas_call(
        functools.partial(_pool_body, S, B),
        grid=(),
        in_specs=[
            pl.BlockSpec(memory_space=pltpu.VMEM),
            pl.BlockSpec(memory_space=pl.ANY),
        ],
        out_specs=pl.BlockSpec(memory_space=pltpu.VMEM),
        out_shape=jax.ShapeDtypeStruct((B, D), jnp.float32),
        scratch_shapes=[pltpu.SemaphoreType.DMA((B,))],
    )


@jax.jit
def kernel(encoded_inputs, input_masks):
    B, S, D = encoded_inputs.shape
    return _build_kernel(B, S, D)(input_masks, encoded_inputs)
